# baseline (speedup 1.0000x reference)
import jax
import jax.numpy as jnp
from jax.experimental import pallas as pl
from jax.experimental.pallas import tpu as pltpu

_B = 20
_T = 500
_N = _B * _T
_G = 20


def _bn(x, g, b, eps=1e-5):
    m = jnp.mean(x, 0)
    v = jnp.var(x, 0)
    return (x - m) / jnp.sqrt(v + eps) * g + b


def _conv1d_same(x, w, b):
    y = jax.lax.conv_general_dilated(x, w, (1,), 'SAME',
                                     dimension_numbers=('NCH', 'OIH', 'NCH'))
    return y + b[None, :, None]


def _gatv2(x, s, d, p):
    n = x.shape[0]
    xl = x @ p['Wl'].T + p['bl']
    xr = x @ p['Wr'].T + p['br']
    e = jax.nn.leaky_relu(xl[s] + xr[d], 0.2)
    a = e @ p['att']
    amax = jax.ops.segment_max(a, d, num_segments=n)
    a = jnp.exp(a - amax[d])
    den = jax.ops.segment_sum(a, d, num_segments=n)
    a = a / (den[d] + 1e-16)
    out = jax.ops.segment_sum(xl[s] * a[:, None], d, num_segments=n)
    return out + p['gat_b']


def _head_kernel(hout_ref, fc1w_ref, fc1b_ref, b1g_ref, b1b_ref,
                 fc2w_ref, fc2b_ref, b2g_ref, b2b_ref,
                 fc3w_ref, fc3b_ref, sig_ref, lsm_ref):
    z = jnp.maximum(hout_ref[...] @ fc1w_ref[...].T + fc1b_ref[...], 0.0)
    m = jnp.mean(z, 0)
    v = jnp.mean((z - m) ** 2, 0)
    z = (z - m) / jnp.sqrt(v + 1e-5) * b1g_ref[...] + b1b_ref[...]
    z = jnp.maximum(z @ fc2w_ref[...].T + fc2b_ref[...], 0.0)
    m = jnp.mean(z, 0)
    v = jnp.mean((z - m) ** 2, 0)
    z = (z - m) / jnp.sqrt(v + 1e-5) * b2g_ref[...] + b2b_ref[...]
    z = z @ fc3w_ref[...].T + fc3b_ref[...]
    sig_ref[...] = jax.nn.sigmoid(z)
    zmax = jnp.max(z, axis=1, keepdims=True)
    ze = z - zmax
    lse = jnp.log(jnp.sum(jnp.exp(ze), axis=1, keepdims=True))
    lsm_ref[...] = ze - lse


def _head(hout, params):
    return pl.pallas_call(
        _head_kernel,
        out_shape=(jax.ShapeDtypeStruct((_G, 10), jnp.float32),
                   jax.ShapeDtypeStruct((_G, 10), jnp.float32)),
    )(hout, params['fc1_w'], params['fc1_b'], params['b1_g'], params['b1_b'],
      params['fc2_w'], params['fc2_b'], params['b2_g'], params['b2_b'],
      params['fc3_w'], params['fc3_b'])


def kernel(data0, data1, edge_index0, batches0, batches1, params):
    src, dst = edge_index0[0], edge_index0[1]
    loop = jnp.arange(_N, dtype=src.dtype)
    s = jnp.concatenate([src, loop])
    d = jnp.concatenate([dst, loop])
    h = data0.reshape(_B, 3, _T)
    for i in range(3):
        p = params['blk%d' % i]
        x = jax.nn.relu(_conv1d_same(h, p['tc1_w'], p['tc1_b']))
        Bs, C, Ts = x.shape
        x = x.reshape(Ts * Bs, C)
        x = jax.nn.relu(_gatv2(x, s, d, p))
        x = _bn(x, p['bn_g'], p['bn_b'])
        x = x.reshape(Bs, x.shape[1], Ts)
        x = jax.nn.relu(_conv1d_same(x, p['tc2_w'], p['tc2_b']))
        h = jax.nn.relu(x)
    hh = data1
    for i in range(4):
        hh = _bn(jax.nn.relu(hh @ params['hW%d' % i].T + params['hb%d' % i]),
                 params['hg%d' % i], params['hbe%d' % i])
    h0 = h.reshape(h.shape[0] * h.shape[2], h.shape[1])
    p0 = jax.ops.segment_sum(h0, batches0, num_segments=_G)
    p1 = jax.ops.segment_sum(hh, batches1, num_segments=_G)
    hout = jnp.concatenate([p0, p1], axis=1)
    sig, lsm = _head(hout, params)
    return (sig, lsm)


# speedup vs baseline: 1.2012x; 1.2012x over previous
import functools

import jax
import jax.numpy as jnp
from jax.experimental import pallas as pl
from jax.experimental.pallas import tpu as pltpu

_B = 20
_T = 500
_N = _B * _T
_G = 20


def _conv_body(x_ref, w_ref, b_ref, o_ref):
    x = x_ref[0]
    w = w_ref[...]
    cin, t = x.shape
    xpad = jnp.concatenate(
        [jnp.zeros((cin, 2), jnp.float32), x, jnp.zeros((cin, 2), jnp.float32)],
        axis=1)
    acc = None
    for k in range(5):
        part = jax.lax.dot_general(
            w[:, :, k], xpad[:, k:k + t],
            (((1,), (0,)), ((), ())),
            preferred_element_type=jnp.float32)
        acc = part if acc is None else acc + part
    acc = acc + b_ref[...][:, None]
    o_ref[0] = jnp.maximum(acc, 0.0)


def _conv1d_relu(x, w, b):
    Bs, Cin, T = x.shape
    Co = w.shape[0]
    return pl.pallas_call(
        _conv_body,
        grid=(Bs,),
        in_specs=[
            pl.BlockSpec((1, Cin, T), lambda i: (i, 0, 0)),
            pl.BlockSpec((Co, Cin, 5), lambda i: (0, 0, 0)),
            pl.BlockSpec((Co,), lambda i: (0,)),
        ],
        out_specs=pl.BlockSpec((1, Co, T), lambda i: (i, 0, 0)),
        out_shape=jax.ShapeDtypeStruct((Bs, Co, T), jnp.float32),
    )(x, w, b)


def _xlxr_body(x_ref, wl_ref, bl_ref, wr_ref, br_ref, xl_ref, xr_ref):
    x = x_ref[...]
    xl_ref[...] = jax.lax.dot_general(
        x, wl_ref[...], (((1,), (1,)), ((), ())),
        preferred_element_type=jnp.float32) + bl_ref[...]
    xr_ref[...] = jax.lax.dot_general(
        x, wr_ref[...], (((1,), (1,)), ((), ())),
        preferred_element_type=jnp.float32) + br_ref[...]


def _xlxr(x, wl, bl, wr, br):
    n, c = x.shape
    rows = 2000
    return pl.pallas_call(
        _xlxr_body,
        grid=(n // rows,),
        in_specs=[
            pl.BlockSpec((rows, c), lambda i: (i, 0)),
            pl.BlockSpec((c, c), lambda i: (0, 0)),
            pl.BlockSpec((c,), lambda i: (0,)),
            pl.BlockSpec((c, c), lambda i: (0, 0)),
            pl.BlockSpec((c,), lambda i: (0,)),
        ],
        out_specs=(pl.BlockSpec((rows, c), lambda i: (i, 0)),
                   pl.BlockSpec((rows, c), lambda i: (i, 0))),
        out_shape=(jax.ShapeDtypeStruct((n, c), jnp.float32),
                   jax.ShapeDtypeStruct((n, c), jnp.float32)),
    )(x, wl, bl, wr, br)


def _post_body(g_ref, gb_ref, bng_ref, bnb_ref, o_ref):
    x = jnp.maximum(g_ref[...] + gb_ref[...], 0.0)
    m = jnp.mean(x, 0)
    v = jnp.mean((x - m) ** 2, 0)
    o_ref[...] = (x - m) / jnp.sqrt(v + 1e-5) * bng_ref[...] + bnb_ref[...]


def _post(gat, gb, bng, bnb):
    n, c = gat.shape
    return pl.pallas_call(
        _post_body,
        out_shape=jax.ShapeDtypeStruct((n, c), jnp.float32),
    )(gat, gb, bng, bnb)


def _mlp1_body(x_ref, *refs):
    x = x_ref[...]
    for i in range(4):
        w, b, g, be = refs[4 * i:4 * i + 4]
        x = jnp.maximum(jax.lax.dot_general(
            x, w[...], (((1,), (1,)), ((), ())),
            preferred_element_type=jnp.float32) + b[...], 0.0)
        m = jnp.mean(x, 0)
        v = jnp.mean((x - m) ** 2, 0)
        x = (x - m) / jnp.sqrt(v + 1e-5) * g[...] + be[...]
    refs[16][...] = x


def _mlp1(data1, params):
    args = [data1]
    for i in range(4):
        args += [params['hW%d' % i], params['hb%d' % i],
                 params['hg%d' % i], params['hbe%d' % i]]
    return pl.pallas_call(
        _mlp1_body,
        out_shape=jax.ShapeDtypeStruct((data1.shape[0], 16), jnp.float32),
    )(*args)


def _head_body(h0_ref, b0_ref, hh_ref, b1_ref,
               fc1w_ref, fc1b_ref, b1g_ref, b1b_ref,
               fc2w_ref, fc2b_ref, b2g_ref, b2b_ref,
               fc3w_ref, fc3b_ref, sig_ref, lsm_ref):
    seg0 = jax.lax.broadcasted_iota(jnp.int32, (_G, _N), 0)
    oh0 = (seg0 == b0_ref[...]).astype(jnp.float32)
    p0 = jax.lax.dot_general(oh0, h0_ref[...], (((1,), (0,)), ((), ())),
                             preferred_element_type=jnp.float32)
    bh = hh_ref[...].shape[0]
    seg1 = jax.lax.broadcasted_iota(jnp.int32, (_G, bh), 0)
    oh1 = (seg1 == b1_ref[...]).astype(jnp.float32)
    p1 = jax.lax.dot_general(oh1, hh_ref[...], (((1,), (0,)), ((), ())),
                             preferred_element_type=jnp.float32)
    z = jnp.concatenate([p0, p1], axis=1)
    z = jnp.maximum(jax.lax.dot_general(
        z, fc1w_ref[...], (((1,), (1,)), ((), ())),
        preferred_element_type=jnp.float32) + fc1b_ref[...], 0.0)
    m = jnp.mean(z, 0)
    v = jnp.mean((z - m) ** 2, 0)
    z = (z - m) / jnp.sqrt(v + 1e-5) * b1g_ref[...] + b1b_ref[...]
    z = jnp.maximum(jax.lax.dot_general(
        z, fc2w_ref[...], (((1,), (1,)), ((), ())),
        preferred_element_type=jnp.float32) + fc2b_ref[...], 0.0)
    m = jnp.mean(z, 0)
    v = jnp.mean((z - m) ** 2, 0)
    z = (z - m) / jnp.sqrt(v + 1e-5) * b2g_ref[...] + b2b_ref[...]
    z = jax.lax.dot_general(z, fc3w_ref[...], (((1,), (1,)), ((), ())),
                            preferred_element_type=jnp.float32) + fc3b_ref[...]
    sig_ref[...] = 1.0 / (1.0 + jnp.exp(-z))
    zmax = jnp.max(z, axis=1, keepdims=True)
    ze = z - zmax
    lse = jnp.log(jnp.sum(jnp.exp(ze), axis=1, keepdims=True))
    lsm_ref[...] = ze - lse


def _head(h0, batches0, hh, batches1, params):
    return pl.pallas_call(
        _head_body,
        out_shape=(jax.ShapeDtypeStruct((_G, 10), jnp.float32),
                   jax.ShapeDtypeStruct((_G, 10), jnp.float32)),
    )(h0, batches0.reshape(1, _N).astype(jnp.int32),
      hh, batches1.reshape(1, -1).astype(jnp.int32),
      params['fc1_w'], params['fc1_b'], params['b1_g'], params['b1_b'],
      params['fc2_w'], params['fc2_b'], params['b2_g'], params['b2_b'],
      params['fc3_w'], params['fc3_b'])


def _edge_stage(xl, xr, s, d, att):
    n = xl.shape[0]
    e = jax.nn.leaky_relu(xl[s] + xr[d], 0.2)
    a = e @ att
    t = jnp.exp(a)
    den = jax.ops.segment_sum(t, d, num_segments=n)
    a = t / den[d]
    return jax.ops.segment_sum(xl[s] * a[:, None], d, num_segments=n)


def kernel(data0, data1, edge_index0, batches0, batches1, params):
    src, dst = edge_index0[0], edge_index0[1]
    loop = jnp.arange(_N, dtype=src.dtype)
    s = jnp.concatenate([src, loop])
    d = jnp.concatenate([dst, loop])
    h = data0.reshape(_B, 3, _T)
    for i in range(3):
        p = params['blk%d' % i]
        x = _conv1d_relu(h, p['tc1_w'], p['tc1_b'])
        Bs, C, Ts = x.shape
        x2 = x.reshape(Ts * Bs, C)
        xl, xr = _xlxr(x2, p['Wl'], p['bl'], p['Wr'], p['br'])
        gat = _edge_stage(xl, xr, s, d, p['att'])
        x2 = _post(gat, p['gat_b'], p['bn_g'], p['bn_b'])
        x = x2.reshape(Bs, C, Ts)
        h = _conv1d_relu(x, p['tc2_w'], p['tc2_b'])
    hh = _mlp1(data1, params)
    h0 = h.reshape(h.shape[0] * h.shape[2], h.shape[1])
    sig, lsm = _head(h0, batches0, hh, batches1, params)
    return (sig, lsm)
